# baseline (speedup 1.0000x reference)
import functools

import jax
import jax.numpy as jnp
from jax import lax
from jax.experimental import pallas as pl
from jax.experimental.pallas import tpu as pltpu
from jax.experimental.pallas import tpu_sc as plsc

N = 10000
K = 32
DF = 128
U = 128

CH = 128
PARTS = 2
NPART = N // PARTS


def _matmul_body(x_ref, w_ref, o_ref):
    o_ref[...] = jnp.dot(x_ref[...].astype(jnp.bfloat16),
                         w_ref[...].astype(jnp.bfloat16),
                         preferred_element_type=jnp.float32)


def _matmul(x, w):
    B = 2000
    return pl.pallas_call(
        _matmul_body,
        grid=(N // B,),
        in_specs=[
            pl.BlockSpec((B, DF), lambda i: (i, 0)),
            pl.BlockSpec((DF, U), lambda i: (0, 0)),
        ],
        out_specs=pl.BlockSpec((B, U), lambda i: (i, 0)),
        out_shape=jax.ShapeDtypeStruct((N, U), jnp.float32),
    )(x, w)


def _sc_gather(table, idx):
    ne = idx.shape[0]
    nchunks = ne // CH
    info = plsc.get_sparse_core_info()
    nc, ns = info.num_cores, info.num_subcores
    nw = nc * ns
    mesh = plsc.VectorSubcoreMesh(core_axis_name="c", subcore_axis_name="s")
    NB = 7
    full = (nchunks // nw) // NB
    rem = nchunks - nw * NB * full

    @functools.partial(
        pl.kernel,
        mesh=mesh,
        out_type=jax.ShapeDtypeStruct((ne, U), jnp.float32),
        scratch_types=(
            [pltpu.VMEM((CH,), jnp.int32)] * NB
            + [pltpu.VMEM((CH, U), jnp.float32)] * NB
            + [pltpu.SemaphoreType.DMA] * (3 * NB)
        ),
    )
    def gk(table_hbm, idx_hbm, out_hbm, *rest):
        ibufs = rest[:NB]
        rbufs = rest[NB:2 * NB]
        isems = rest[2 * NB:3 * NB]
        gsems = rest[3 * NB:4 * NB]
        wsems = rest[4 * NB:5 * NB]
        wid = lax.axis_index("s") * nc + lax.axis_index("c")

        def run_block(offs):
            cps = [pltpu.async_copy(idx_hbm.at[pl.ds(o, CH)], ibufs[j],
                                    isems[j]) for j, o in enumerate(offs)]
            gs = []
            for j, o in enumerate(offs):
                cps[j].wait()
                gs.append(pltpu.async_copy(table_hbm.at[ibufs[j]], rbufs[j],
                                           gsems[j]))
            ws = []
            for j, o in enumerate(offs):
                gs[j].wait()
                ws.append(pltpu.async_copy(rbufs[j],
                                           out_hbm.at[pl.ds(o, CH)],
                                           wsems[j]))
            for w_ in ws:
                w_.wait()

        def body(s, carry):
            run_block([(wid + (NB * s + j) * nw) * CH for j in range(NB)])
            return carry

        lax.fori_loop(0, full, body, 0)

        nfull_tail = rem // nw
        extra = rem - nfull_tail * nw
        base = full * NB * nw
        if nfull_tail:
            run_block([(base + wid + j * nw) * CH for j in range(nfull_tail)])

        @pl.when(wid < extra)
        def _():
            off = (base + nfull_tail * nw + wid) * CH
            pltpu.sync_copy(idx_hbm.at[pl.ds(off, CH)], ibufs[0])
            pltpu.async_copy(table_hbm.at[ibufs[0]], rbufs[0], gsems[0]).wait()
            pltpu.sync_copy(rbufs[0], out_hbm.at[pl.ds(off, CH)])

    return gk(table, idx)


def _batcher_pairs(n):
    pairs = []
    p = 1
    while p < n:
        k = p
        while k >= 1:
            for j in range(k % p, n - k, 2 * k):
                for i in range(min(k, n - j - k)):
                    if (i + j) // (2 * p) == (i + j + k) // (2 * p):
                        pairs.append((i + j, i + j + k))
            k //= 2
        p *= 2
    return pairs


_PAIRS16 = _batcher_pairs(16)


def _sort16(vals):
    vals = list(vals)
    for a, b in _PAIRS16:
        lo = jnp.minimum(vals[a], vals[b])
        hi = jnp.maximum(vals[a], vals[b])
        vals[a], vals[b] = lo, hi
    return vals


def _median32(vals):
    a = _sort16(vals[:16])
    b = _sort16(vals[16:])
    lo = [jnp.minimum(a[i], b[15 - i]) for i in range(16)]
    hi = [jnp.maximum(a[i], b[15 - i]) for i in range(16)]
    mx = functools.reduce(jnp.maximum, lo)
    mn = functools.reduce(jnp.minimum, hi)
    return (mx.astype(jnp.float32) + mn.astype(jnp.float32)) * 0.5


def _median_body(msg_ref, o_ref):
    vals = [msg_ref[k].astype(jnp.bfloat16) for k in range(K)]
    o_ref[...] = _median32(vals)


def _median(msg):
    n = msg.shape[1]
    B = 200
    assert n % B == 0
    return pl.pallas_call(
        _median_body,
        grid=(n // B,),
        in_specs=[pl.BlockSpec((K, B, U), lambda i: (0, i, 0))],
        out_specs=pl.BlockSpec((B, U), lambda i: (i, 0)),
        out_shape=jax.ShapeDtypeStruct((n, U), jnp.float32),
    )(msg)


def kernel(x, neighbors, kernel):
    w = kernel
    h = _matmul(x, w)
    nbt = neighbors.astype(jnp.int32).T
    outs = []
    for p in range(PARTS):
        idx = lax.slice_in_dim(nbt, p * NPART, (p + 1) * NPART,
                               axis=1).reshape(-1)
        msg = _sc_gather(h, idx)
        outs.append(_median(msg.reshape(K, NPART, U)))
    return jnp.concatenate(outs, axis=0)


# speedup vs baseline: 1.0607x; 1.0607x over previous
import functools

import jax
import jax.numpy as jnp
from jax import lax
from jax.experimental import pallas as pl
from jax.experimental.pallas import tpu as pltpu
from jax.experimental.pallas import tpu_sc as plsc

N = 10000
K = 32
DF = 128
U = 128

E = N * K
CH = 128
NCHUNKS = E // CH


def _sc_gather(table, idx):
    info = plsc.get_sparse_core_info()
    nc, ns = info.num_cores, info.num_subcores
    nw = nc * ns
    mesh = plsc.VectorSubcoreMesh(core_axis_name="c", subcore_axis_name="s")
    NB = 7
    full = (NCHUNKS // nw) // NB
    rem = NCHUNKS - nw * NB * full

    @functools.partial(
        pl.kernel,
        mesh=mesh,
        out_type=jax.ShapeDtypeStruct((E, DF), jnp.float32),
        scratch_types=(
            [pltpu.VMEM((CH,), jnp.int32)] * NB
            + [pltpu.VMEM((CH, DF), jnp.float32)] * NB
            + [pltpu.SemaphoreType.DMA] * (3 * NB)
        ),
    )
    def gk(table_hbm, idx_hbm, out_hbm, *rest):
        ibufs = rest[:NB]
        rbufs = rest[NB:2 * NB]
        isems = rest[2 * NB:3 * NB]
        gsems = rest[3 * NB:4 * NB]
        wsems = rest[4 * NB:5 * NB]
        wid = lax.axis_index("s") * nc + lax.axis_index("c")

        def run_block(offs):
            cps = [pltpu.async_copy(idx_hbm.at[pl.ds(o, CH)], ibufs[j],
                                    isems[j]) for j, o in enumerate(offs)]
            gs = []
            for j, o in enumerate(offs):
                cps[j].wait()
                gs.append(pltpu.async_copy(table_hbm.at[ibufs[j]], rbufs[j],
                                           gsems[j]))
            ws = []
            for j, o in enumerate(offs):
                gs[j].wait()
                ws.append(pltpu.async_copy(rbufs[j],
                                           out_hbm.at[pl.ds(o, CH)],
                                           wsems[j]))
            for w_ in ws:
                w_.wait()

        def body(s, carry):
            run_block([(wid + (NB * s + j) * nw) * CH for j in range(NB)])
            return carry

        lax.fori_loop(0, full, body, 0)

        nfull_tail = rem // nw
        extra = rem - nfull_tail * nw
        base = full * NB * nw
        if nfull_tail:
            run_block([(base + wid + j * nw) * CH for j in range(nfull_tail)])

        @pl.when(wid < extra)
        def _():
            off = (base + nfull_tail * nw + wid) * CH
            pltpu.sync_copy(idx_hbm.at[pl.ds(off, CH)], ibufs[0])
            pltpu.async_copy(table_hbm.at[ibufs[0]], rbufs[0], gsems[0]).wait()
            pltpu.sync_copy(rbufs[0], out_hbm.at[pl.ds(off, CH)])

    return gk(table, idx)


def _batcher_pairs(n):
    pairs = []
    p = 1
    while p < n:
        k = p
        while k >= 1:
            for j in range(k % p, n - k, 2 * k):
                for i in range(min(k, n - j - k)):
                    if (i + j) // (2 * p) == (i + j + k) // (2 * p):
                        pairs.append((i + j, i + j + k))
            k //= 2
        p *= 2
    return pairs


_PAIRS16 = _batcher_pairs(16)


def _sort16(vals):
    vals = list(vals)
    for a, b in _PAIRS16:
        lo = jnp.minimum(vals[a], vals[b])
        hi = jnp.maximum(vals[a], vals[b])
        vals[a], vals[b] = lo, hi
    return vals


def _median32(vals):
    a = _sort16(vals[:16])
    b = _sort16(vals[16:])
    lo = [jnp.minimum(a[i], b[15 - i]) for i in range(16)]
    hi = [jnp.maximum(a[i], b[15 - i]) for i in range(16)]
    mx = functools.reduce(jnp.maximum, lo)
    mn = functools.reduce(jnp.minimum, hi)
    return (mx.astype(jnp.float32) + mn.astype(jnp.float32)) * 0.5


_MB = 400


def _median_body(msg_ref, w_ref, o_ref):
    r = msg_ref[...].reshape(K * _MB, DF).astype(jnp.bfloat16)
    h = jnp.dot(r, w_ref[...].astype(jnp.bfloat16),
                preferred_element_type=jnp.float32).astype(jnp.bfloat16)
    h = h.reshape(K, _MB, U)
    o_ref[...] = _median32([h[k] for k in range(K)])


def _median(msg, w):
    return pl.pallas_call(
        _median_body,
        grid=(N // _MB,),
        in_specs=[
            pl.BlockSpec((K, _MB, DF), lambda i: (0, i, 0)),
            pl.BlockSpec((DF, U), lambda i: (0, 0)),
        ],
        out_specs=pl.BlockSpec((_MB, U), lambda i: (i, 0)),
        out_shape=jax.ShapeDtypeStruct((N, U), jnp.float32),
    )(msg, w)


def kernel(x, neighbors, kernel):
    w = kernel
    idx = neighbors.astype(jnp.int32).T.reshape(-1)
    msgx = _sc_gather(x, idx)
    return _median(msgx.reshape(K, N, DF), w)


# speedup vs baseline: 1.0774x; 1.0157x over previous
import functools

import jax
import jax.numpy as jnp
from jax import lax
from jax.experimental import pallas as pl
from jax.experimental.pallas import tpu as pltpu
from jax.experimental.pallas import tpu_sc as plsc

N = 10000
K = 32
DF = 128
U = 128

E = N * K
CH = 128
NCHUNKS = E // CH


def _matmul_body(x_ref, w_ref, o_ref):
    o_ref[...] = jnp.dot(x_ref[...], w_ref[...],
                         preferred_element_type=jnp.float32)


def _matmul(x, w):
    B = 2000
    return pl.pallas_call(
        _matmul_body,
        grid=(N // B,),
        in_specs=[
            pl.BlockSpec((B, DF), lambda i: (i, 0)),
            pl.BlockSpec((DF, U), lambda i: (0, 0)),
        ],
        out_specs=pl.BlockSpec((B, U), lambda i: (i, 0)),
        out_shape=jax.ShapeDtypeStruct((N, U), jnp.float32),
    )(x, w)


def _sc_gather(table, idx):
    info = plsc.get_sparse_core_info()
    nc, ns = info.num_cores, info.num_subcores
    nw = nc * ns
    mesh = plsc.VectorSubcoreMesh(core_axis_name="c", subcore_axis_name="s")
    NB = 6
    full = (NCHUNKS // nw) // NB
    rem = NCHUNKS - nw * NB * full

    @functools.partial(
        pl.kernel,
        mesh=mesh,
        out_type=jax.ShapeDtypeStruct((E, U), jnp.float32),
        scratch_types=(
            [pltpu.VMEM((CH,), jnp.int32)] * NB
            + [pltpu.VMEM((CH, U), jnp.float32)] * NB
            + [pltpu.SemaphoreType.DMA] * (3 * NB)
        ),
    )
    def gk(table_hbm, idx_hbm, out_hbm, *rest):
        ibufs = rest[:NB]
        rbufs = rest[NB:2 * NB]
        isems = rest[2 * NB:3 * NB]
        gsems = rest[3 * NB:4 * NB]
        wsems = rest[4 * NB:5 * NB]
        wid = lax.axis_index("s") * nc + lax.axis_index("c")

        def run_block(offs):
            cps = [pltpu.async_copy(idx_hbm.at[pl.ds(o, CH)], ibufs[j],
                                    isems[j]) for j, o in enumerate(offs)]
            gs = []
            for j, o in enumerate(offs):
                cps[j].wait()
                gs.append(pltpu.async_copy(table_hbm.at[ibufs[j]], rbufs[j],
                                           gsems[j]))
            ws = []
            for j, o in enumerate(offs):
                gs[j].wait()
                ws.append(pltpu.async_copy(rbufs[j],
                                           out_hbm.at[pl.ds(o, CH)],
                                           wsems[j]))
            for w_ in ws:
                w_.wait()

        def body(s, carry):
            run_block([(wid + (NB * s + j) * nw) * CH for j in range(NB)])
            return carry

        lax.fori_loop(0, full, body, 0)

        nfull_tail = rem // nw
        extra = rem - nfull_tail * nw
        base = full * NB * nw
        if nfull_tail:
            run_block([(base + wid + j * nw) * CH for j in range(nfull_tail)])

        @pl.when(wid < extra)
        def _():
            off = (base + nfull_tail * nw + wid) * CH
            pltpu.sync_copy(idx_hbm.at[pl.ds(off, CH)], ibufs[0])
            pltpu.async_copy(table_hbm.at[ibufs[0]], rbufs[0], gsems[0]).wait()
            pltpu.sync_copy(rbufs[0], out_hbm.at[pl.ds(off, CH)])

    return gk(table, idx)


def _batcher_pairs(n):
    pairs = []
    p = 1
    while p < n:
        k = p
        while k >= 1:
            for j in range(k % p, n - k, 2 * k):
                for i in range(min(k, n - j - k)):
                    if (i + j) // (2 * p) == (i + j + k) // (2 * p):
                        pairs.append((i + j, i + j + k))
            k //= 2
        p *= 2
    return pairs


_PAIRS16 = _batcher_pairs(16)


def _sort16(vals):
    vals = list(vals)
    for a, b in _PAIRS16:
        lo = jnp.minimum(vals[a], vals[b])
        hi = jnp.maximum(vals[a], vals[b])
        vals[a], vals[b] = lo, hi
    return vals


def _median32(vals):
    a = _sort16(vals[:16])
    b = _sort16(vals[16:])
    lo = [jnp.minimum(a[i], b[15 - i]) for i in range(16)]
    hi = [jnp.maximum(a[i], b[15 - i]) for i in range(16)]
    mx = functools.reduce(jnp.maximum, lo)
    mn = functools.reduce(jnp.minimum, hi)
    return (mx.astype(jnp.float32) + mn.astype(jnp.float32)) * 0.5


def _median_body(msg_ref, o_ref):
    vals = [msg_ref[k].astype(jnp.bfloat16) for k in range(K)]
    o_ref[...] = _median32(vals)


def _median(msg):
    B = 400
    return pl.pallas_call(
        _median_body,
        grid=(N // B,),
        in_specs=[pl.BlockSpec((K, B, U), lambda i: (0, i, 0))],
        out_specs=pl.BlockSpec((B, U), lambda i: (i, 0)),
        out_shape=jax.ShapeDtypeStruct((N, U), jnp.float32),
    )(msg)


def kernel(x, neighbors, kernel):
    w = kernel
    h = _matmul(x, w)
    idx = neighbors.astype(jnp.int32).T.reshape(-1)
    msg = _sc_gather(h, idx)
    return _median(msg.reshape(K, N, U))


# speedup vs baseline: 1.0959x; 1.0172x over previous
import functools

import jax
import jax.numpy as jnp
from jax import lax
from jax.experimental import pallas as pl
from jax.experimental.pallas import tpu as pltpu
from jax.experimental.pallas import tpu_sc as plsc

N = 10000
K = 32
DF = 128
U = 128

E = N * K
CH = 128
NCHUNKS = E // CH


def _matmul_body(x_ref, w_ref, o_ref):
    o_ref[...] = jnp.dot(x_ref[...], w_ref[...],
                         preferred_element_type=jnp.float32)


def _matmul(x, w):
    B = 2000
    return pl.pallas_call(
        _matmul_body,
        grid=(N // B,),
        in_specs=[
            pl.BlockSpec((B, DF), lambda i: (i, 0)),
            pl.BlockSpec((DF, U), lambda i: (0, 0)),
        ],
        out_specs=pl.BlockSpec((B, U), lambda i: (i, 0)),
        out_shape=jax.ShapeDtypeStruct((N, U), jnp.float32),
    )(x, w)


def _sc_gather(table, idx):
    info = plsc.get_sparse_core_info()
    nc, ns = info.num_cores, info.num_subcores
    nw = nc * ns
    mesh = plsc.VectorSubcoreMesh(core_axis_name="c", subcore_axis_name="s")
    NB = 6
    full = (NCHUNKS // nw) // NB
    rem = NCHUNKS - nw * NB * full

    @functools.partial(
        pl.kernel,
        mesh=mesh,
        out_type=jax.ShapeDtypeStruct((E, U), jnp.float32),
        scratch_types=(
            [pltpu.VMEM((CH,), jnp.int32)] * NB
            + [pltpu.VMEM((CH, U), jnp.float32)] * NB
            + [pltpu.SemaphoreType.DMA] * (3 * NB)
        ),
    )
    def gk(table_hbm, idx_hbm, out_hbm, *rest):
        ibufs = rest[:NB]
        rbufs = rest[NB:2 * NB]
        isems = rest[2 * NB:3 * NB]
        gsems = rest[3 * NB:4 * NB]
        wsems = rest[4 * NB:5 * NB]
        wid = lax.axis_index("s") * nc + lax.axis_index("c")

        def run_block(offs):
            cps = [pltpu.async_copy(idx_hbm.at[pl.ds(o, CH)], ibufs[j],
                                    isems[j]) for j, o in enumerate(offs)]
            gs = []
            for j, o in enumerate(offs):
                cps[j].wait()
                gs.append(pltpu.async_copy(table_hbm.at[ibufs[j]], rbufs[j],
                                           gsems[j]))
            ws = []
            for j, o in enumerate(offs):
                gs[j].wait()
                ws.append(pltpu.async_copy(rbufs[j],
                                           out_hbm.at[pl.ds(o, CH)],
                                           wsems[j]))
            for w_ in ws:
                w_.wait()

        def body(s, carry):
            run_block([(wid + (NB * s + j) * nw) * CH for j in range(NB)])
            return carry

        lax.fori_loop(0, full, body, 0)

        nfull_tail = rem // nw
        extra = rem - nfull_tail * nw
        base = full * NB * nw
        if nfull_tail:
            run_block([(base + wid + j * nw) * CH for j in range(nfull_tail)])

        @pl.when(wid < extra)
        def _():
            off = (base + nfull_tail * nw + wid) * CH
            pltpu.sync_copy(idx_hbm.at[pl.ds(off, CH)], ibufs[0])
            pltpu.async_copy(table_hbm.at[ibufs[0]], rbufs[0], gsems[0]).wait()
            pltpu.sync_copy(rbufs[0], out_hbm.at[pl.ds(off, CH)])

    return gk(table, idx)


def _batcher_pairs(n):
    pairs = []
    p = 1
    while p < n:
        k = p
        while k >= 1:
            for j in range(k % p, n - k, 2 * k):
                for i in range(min(k, n - j - k)):
                    if (i + j) // (2 * p) == (i + j + k) // (2 * p):
                        pairs.append((i + j, i + j + k))
            k //= 2
        p *= 2
    return pairs


_PAIRS16 = _batcher_pairs(16)


def _sort16(vals):
    vals = list(vals)
    for a, b in _PAIRS16:
        lo = jnp.minimum(vals[a], vals[b])
        hi = jnp.maximum(vals[a], vals[b])
        vals[a], vals[b] = lo, hi
    return vals


def _median32(vals):
    a = _sort16(vals[:16])
    b = _sort16(vals[16:])
    lo = [jnp.minimum(a[i], b[15 - i]) for i in range(16)]
    hi = [jnp.maximum(a[i], b[15 - i]) for i in range(16)]
    mx = functools.reduce(jnp.maximum, lo)
    mn = functools.reduce(jnp.minimum, hi)
    return (mx.astype(jnp.float32) + mn.astype(jnp.float32)) * 0.5


def _median_body(msg_ref, o_ref):
    vals = [msg_ref[k].astype(jnp.bfloat16) for k in range(K)]
    o_ref[...] = _median32(vals)


def _median(msg):
    B = 1000
    return pl.pallas_call(
        _median_body,
        grid=(N // B,),
        in_specs=[pl.BlockSpec((K, B, U), lambda i: (0, i, 0))],
        out_specs=pl.BlockSpec((B, U), lambda i: (i, 0)),
        out_shape=jax.ShapeDtypeStruct((N, U), jnp.float32),
    )(msg)


def kernel(x, neighbors, kernel):
    w = kernel
    h = _matmul(x, w)
    idx = neighbors.astype(jnp.int32).T.reshape(-1)
    msg = _sc_gather(h, idx)
    return _median(msg.reshape(K, N, U))
